# baseline (speedup 1.0000x reference)
import functools

import jax
import jax.numpy as jnp
from jax import lax
from jax.experimental import pallas as pl
from jax.experimental.pallas import tpu as pltpu
from jax.experimental.pallas import tpu_sc as plsc

PROMPT_LEN = 100
EMBED = 64
RPC = 4
NBUF = 2


def kernel(input, prompt_table, normal_table):
    B, S = input.shape
    assert S == 2 * PROMPT_LEN
    info = plsc.get_sparse_core_info()
    num_workers = info.num_cores * info.num_subcores
    rows_per_w = B // num_workers
    nchunks = rows_per_w // RPC

    mesh = plsc.VectorSubcoreMesh(core_axis_name="c", subcore_axis_name="s")

    @functools.partial(
        pl.kernel,
        out_type=jax.ShapeDtypeStruct((B * S, EMBED), jnp.float32),
        mesh=mesh,
        scratch_types=[
            pltpu.VMEM((rows_per_w, 2, PROMPT_LEN), jnp.int32),
            [pltpu.VMEM((RPC * S, EMBED), jnp.float32) for _ in range(NBUF)],
            [pltpu.SemaphoreType.DMA for _ in range(NBUF)],
            [pltpu.SemaphoreType.DMA for _ in range(NBUF)],
        ],
        compiler_params=pltpu.CompilerParams(use_tc_tiling_on_sc=False),
    )
    def emb(inp_hbm, ptab_hbm, ntab_hbm, out_hbm, idx_v, rows_v, gsems, ssems):
        wid = lax.axis_index("s") * info.num_cores + lax.axis_index("c")
        row0 = wid * rows_per_w
        out0 = row0 * S

        pltpu.sync_copy(inp_hbm.at[pl.ds(row0, rows_per_w)], idx_v)

        def fire_gathers(c, b):
            for r in range(RPC):
                row = c * RPC + r
                pltpu.async_copy(
                    ptab_hbm.at[idx_v.at[row, 0]],
                    rows_v[b].at[pl.ds(r * S, PROMPT_LEN)],
                    gsems[b],
                )
                pltpu.async_copy(
                    ntab_hbm.at[idx_v.at[row, 1]],
                    rows_v[b].at[pl.ds(r * S + PROMPT_LEN, PROMPT_LEN)],
                    gsems[b],
                )

        def wait_gathers(b):
            for r in range(RPC):
                pltpu.make_async_copy(
                    ptab_hbm.at[idx_v.at[0, 0]],
                    rows_v[b].at[pl.ds(r * S, PROMPT_LEN)],
                    gsems[b],
                ).wait()
                pltpu.make_async_copy(
                    ntab_hbm.at[idx_v.at[0, 1]],
                    rows_v[b].at[pl.ds(r * S + PROMPT_LEN, PROMPT_LEN)],
                    gsems[b],
                ).wait()

        def fire_store(c, b):
            pltpu.async_copy(
                rows_v[b], out_hbm.at[pl.ds(out0 + c * (RPC * S), RPC * S)], ssems[b]
            )

        def wait_store(b):
            pltpu.make_async_copy(
                rows_v[b], out_hbm.at[pl.ds(out0, RPC * S)], ssems[b]
            ).wait()

        for b in range(NBUF):
            fire_gathers(b, b)

        def body(g, carry):
            for b in range(NBUF):
                c = g * NBUF + b
                wait_gathers(b)
                fire_store(c, b)
            for b in range(NBUF):
                c = g * NBUF + b
                wait_store(b)
                fire_gathers(c + NBUF, b)
            return carry

        lax.fori_loop(0, nchunks // NBUF - 1, body, 0)

        for b in range(NBUF):
            c = nchunks - NBUF + b
            wait_gathers(b)
            fire_store(c, b)
        for b in range(NBUF):
            wait_store(b)

    inp3 = input.reshape(B, 2, PROMPT_LEN)
    out = emb(inp3, prompt_table, normal_table)
    return out.reshape(B, S, EMBED)


# speedup vs baseline: 1.0021x; 1.0021x over previous
import functools

import jax
import jax.numpy as jnp
from jax import lax
from jax.experimental import pallas as pl
from jax.experimental.pallas import tpu as pltpu
from jax.experimental.pallas import tpu_sc as plsc

PROMPT_LEN = 100
EMBED = 64
RPC = 2
NBUF = 4


def kernel(input, prompt_table, normal_table):
    B, S = input.shape
    assert S == 2 * PROMPT_LEN
    info = plsc.get_sparse_core_info()
    num_workers = info.num_cores * info.num_subcores
    rows_per_w = B // num_workers
    nchunks = rows_per_w // RPC

    mesh = plsc.VectorSubcoreMesh(core_axis_name="c", subcore_axis_name="s")

    @functools.partial(
        pl.kernel,
        out_type=jax.ShapeDtypeStruct((B * S, EMBED), jnp.float32),
        mesh=mesh,
        scratch_types=[
            pltpu.VMEM((rows_per_w, 2, PROMPT_LEN), jnp.int32),
            [pltpu.VMEM((RPC * S, EMBED), jnp.float32) for _ in range(NBUF)],
            [pltpu.SemaphoreType.DMA for _ in range(NBUF)],
            [pltpu.SemaphoreType.DMA for _ in range(NBUF)],
        ],
        compiler_params=pltpu.CompilerParams(use_tc_tiling_on_sc=False),
    )
    def emb(inp_hbm, ptab_hbm, ntab_hbm, out_hbm, idx_v, rows_v, gsems, ssems):
        wid = lax.axis_index("s") * info.num_cores + lax.axis_index("c")
        row0 = wid * rows_per_w
        out0 = row0 * S

        pltpu.sync_copy(inp_hbm.at[pl.ds(row0, rows_per_w)], idx_v)

        def fire_gathers(c, b):
            for r in range(RPC):
                row = c * RPC + r
                pltpu.async_copy(
                    ptab_hbm.at[idx_v.at[row, 0]],
                    rows_v[b].at[pl.ds(r * S, PROMPT_LEN)],
                    gsems[b],
                )
                pltpu.async_copy(
                    ntab_hbm.at[idx_v.at[row, 1]],
                    rows_v[b].at[pl.ds(r * S + PROMPT_LEN, PROMPT_LEN)],
                    gsems[b],
                )

        def wait_gathers(b):
            for r in range(RPC):
                pltpu.make_async_copy(
                    ptab_hbm.at[idx_v.at[0, 0]],
                    rows_v[b].at[pl.ds(r * S, PROMPT_LEN)],
                    gsems[b],
                ).wait()
                pltpu.make_async_copy(
                    ntab_hbm.at[idx_v.at[0, 1]],
                    rows_v[b].at[pl.ds(r * S + PROMPT_LEN, PROMPT_LEN)],
                    gsems[b],
                ).wait()

        def fire_store(c, b):
            pltpu.async_copy(
                rows_v[b], out_hbm.at[pl.ds(out0 + c * (RPC * S), RPC * S)], ssems[b]
            )

        def wait_store(b):
            pltpu.make_async_copy(
                rows_v[b], out_hbm.at[pl.ds(out0, RPC * S)], ssems[b]
            ).wait()

        for b in range(NBUF):
            fire_gathers(b, b)

        def body(g, carry):
            for b in range(NBUF):
                c = g * NBUF + b
                wait_gathers(b)
                fire_store(c, b)
            for b in range(NBUF):
                c = g * NBUF + b
                wait_store(b)
                fire_gathers(c + NBUF, b)
            return carry

        lax.fori_loop(0, nchunks // NBUF - 1, body, 0)

        for b in range(NBUF):
            c = nchunks - NBUF + b
            wait_gathers(b)
            fire_store(c, b)
        for b in range(NBUF):
            wait_store(b)

    inp3 = input.reshape(B, 2, PROMPT_LEN)
    out = emb(inp3, prompt_table, normal_table)
    return out.reshape(B, S, EMBED)
